# Initial kernel scaffold; baseline (speedup 1.0000x reference)
import jax
import jax.numpy as jnp
from jax.experimental import pallas as pl


def kernel(coords, bond_indices, angle_indices):
    raise NotImplementedError("write your pallas kernel here")


# speedup vs baseline: 15.6509x; 15.6509x over previous
import jax
import jax.numpy as jnp
from jax.experimental import pallas as pl
from jax.experimental.pallas import tpu as pltpu

_NB = 4096
_NA = 8192
_F = 1024


def _pc_body(bi_ref, ai_ref, table_ref, out_ref):
    zero = jnp.zeros((8, 128), jnp.float32)

    def bond_step(k, carry):
        s, ss = carry
        i = bi_ref[2 * k]
        j = bi_ref[2 * k + 1]
        a1 = table_ref[i]
        a2 = table_ref[j]
        d = a1 - a2
        d2 = d * d
        l2 = d2[0:8] + d2[8:16] + d2[16:24]
        l = jnp.sqrt(l2)
        return (s + l, ss + l2)

    bsum, bsq = jax.lax.fori_loop(0, _NB, bond_step, (zero, zero), unroll=8)

    def angle_step(k, carry):
        s, ss = carry
        i1 = ai_ref[3 * k]
        i2 = ai_ref[3 * k + 1]
        i3 = ai_ref[3 * k + 2]
        a1 = table_ref[i1]
        a2 = table_ref[i2]
        a3 = table_ref[i3]
        v1 = a1 - a2
        v2 = a3 - a2
        p11 = v1 * v1
        p22 = v2 * v2
        p12 = v1 * v2
        n1sq = p11[0:8] + p11[8:16] + p11[16:24]
        n2sq = p22[0:8] + p22[8:16] + p22[16:24]
        dot = p12[0:8] + p12[8:16] + p12[16:24]
        n1 = jnp.maximum(jnp.sqrt(n1sq), 1e-12)
        n2 = jnp.maximum(jnp.sqrt(n2sq), 1e-12)
        cos = jnp.clip(dot / (n1 * n2), -1.0, 1.0)
        return (s + cos, ss + cos * cos)

    asum, asq = jax.lax.fori_loop(0, _NA, angle_step, (zero, zero), unroll=8)

    bvar = (bsq - bsum * bsum * (1.0 / _NB)) * (1.0 / (_NB - 1))
    avar = (asq - asum * asum * (1.0 / _NA)) * (1.0 / (_NA - 1))
    out_ref[0, 0] = (jnp.sum(bvar) + jnp.sum(avar)) * (1.0 / _F)


def kernel(coords, bond_indices, angle_indices):
    bi = bond_indices.astype(jnp.int32).reshape(-1)
    ai = angle_indices.astype(jnp.int32).reshape(-1)
    table = jnp.transpose(coords.reshape(8, 128, 2048, 3), (2, 3, 0, 1))
    table = table.reshape(2048, 24, 128)
    out = pl.pallas_call(
        _pc_body,
        out_shape=jax.ShapeDtypeStruct((1, 1), jnp.float32),
        in_specs=[
            pl.BlockSpec(memory_space=pltpu.SMEM),
            pl.BlockSpec(memory_space=pltpu.SMEM),
            pl.BlockSpec(memory_space=pltpu.VMEM),
        ],
        out_specs=pl.BlockSpec(memory_space=pltpu.SMEM),
    )(bi, ai, table)
    return out[0, 0]
